# baseline (speedup 1.0000x reference)
import jax
import jax.numpy as jnp
from jax.experimental import pallas as pl
from jax.experimental.pallas import tpu as pltpu


def _tile4_kernel(x_ref, o_ref):
    xb = x_ref[...]
    d = xb.shape[1]
    o_ref[:, 0 * d:1 * d] = xb
    o_ref[:, 1 * d:2 * d] = xb
    o_ref[:, 2 * d:3 * d] = xb
    o_ref[:, 3 * d:4 * d] = xb


def kernel(x, edge_index):
    del edge_index
    n, d = x.shape
    bn = 10000
    if n % bn != 0 or bn % 8 != 0:
        bn = n
    out = pl.pallas_call(
        _tile4_kernel,
        grid=(n // bn,),
        in_specs=[pl.BlockSpec((bn, d), lambda i: (i, 0))],
        out_specs=pl.BlockSpec((bn, 4 * d), lambda i: (i, 0)),
        out_shape=jax.ShapeDtypeStruct((n, 4 * d), x.dtype),
        compiler_params=pltpu.CompilerParams(
            dimension_semantics=("parallel",)),
    )(x)
    return out


# speedup vs baseline: 1.1453x; 1.1453x over previous
import jax
import jax.numpy as jnp
from jax.experimental import pallas as pl
from jax.experimental.pallas import tpu as pltpu


def _dma_tile4_kernel(x_hbm, o_hbm, vbuf, in_sem, out_sems):
    d = vbuf.shape[1]
    in_cp = pltpu.make_async_copy(x_hbm, vbuf, in_sem)
    in_cp.start()
    in_cp.wait()
    cps = []
    for j in range(4):
        cp = pltpu.make_async_copy(
            vbuf, o_hbm.at[:, pl.ds(j * d, d)], out_sems.at[j])
        cp.start()
        cps.append(cp)
    for cp in cps:
        cp.wait()


def kernel(x, edge_index):
    del edge_index
    n, d = x.shape
    out = pl.pallas_call(
        _dma_tile4_kernel,
        in_specs=[pl.BlockSpec(memory_space=pl.ANY)],
        out_specs=pl.BlockSpec(memory_space=pl.ANY),
        out_shape=jax.ShapeDtypeStruct((n, 4 * d), x.dtype),
        scratch_shapes=[
            pltpu.VMEM((n, d), x.dtype),
            pltpu.SemaphoreType.DMA,
            pltpu.SemaphoreType.DMA((4,)),
        ],
    )(x)
    return out


# speedup vs baseline: 1.2137x; 1.0597x over previous
import jax
import jax.numpy as jnp
from jax.experimental import pallas as pl
from jax.experimental.pallas import tpu as pltpu


_CHUNKS = 2


def _dma_tile4_kernel(x_hbm, o_hbm, vbuf, in_sems, out_sems):
    n, d = vbuf.shape
    h = n // _CHUNKS
    in_cps = []
    for c in range(_CHUNKS):
        rows = pl.ds(c * h, h)
        cp = pltpu.make_async_copy(
            x_hbm.at[rows, :], vbuf.at[rows, :], in_sems.at[c])
        cp.start()
        in_cps.append(cp)
    out_cps = []
    for c in range(_CHUNKS):
        in_cps[c].wait()
        rows = pl.ds(c * h, h)
        for j in range(4):
            cp = pltpu.make_async_copy(
                vbuf.at[rows, :], o_hbm.at[rows, pl.ds(j * d, d)],
                out_sems.at[c, j])
            cp.start()
            out_cps.append(cp)
    for cp in out_cps:
        cp.wait()


def kernel(x, edge_index):
    del edge_index
    n, d = x.shape
    out = pl.pallas_call(
        _dma_tile4_kernel,
        in_specs=[pl.BlockSpec(memory_space=pl.ANY)],
        out_specs=pl.BlockSpec(memory_space=pl.ANY),
        out_shape=jax.ShapeDtypeStruct((n, 4 * d), x.dtype),
        scratch_shapes=[
            pltpu.VMEM((n, d), x.dtype),
            pltpu.SemaphoreType.DMA((_CHUNKS,)),
            pltpu.SemaphoreType.DMA((_CHUNKS, 4)),
        ],
    )(x)
    return out
